# baseline (speedup 1.0000x reference)
import jax
import jax.numpy as jnp
from jax.experimental import pallas as pl

N_RATE = 6
N_GENRE = 25
N_DIRECTOR = 2186
N_YEAR = 81
EMB = 32
D = 2 + N_GENRE + N_DIRECTOR
BT = 2048


def _tile_kernel(x_ref, w_big_ref, w_rate_ref, w_year_ref, out_ref):
    xf = x_ref[...].astype(jnp.float32)
    big = jax.lax.dot_general(
        xf, w_big_ref[...],
        (((1,), (0,)), ((), ())),
        preferred_element_type=jnp.float32,
    )
    gd = jax.nn.sigmoid(big)

    rate_idx = x_ref[:, 0:1]
    year_idx = x_ref[:, 1:2]
    oh_rate = (rate_idx == jax.lax.broadcasted_iota(jnp.int32, (1, N_RATE), 1)
               ).astype(jnp.float32)
    oh_year = (year_idx == jax.lax.broadcasted_iota(jnp.int32, (1, N_YEAR), 1)
               ).astype(jnp.float32)
    rate_emb = jax.lax.dot_general(
        oh_rate, w_rate_ref[...], (((1,), (0,)), ((), ())),
        preferred_element_type=jnp.float32)
    year_emb = jax.lax.dot_general(
        oh_year, w_year_ref[...], (((1,), (0,)), ((), ())),
        preferred_element_type=jnp.float32)

    out_ref[...] = jnp.concatenate([rate_emb, year_emb, gd], axis=1)


def kernel(x, W_rate, W_year, W_genre, W_director):
    B = x.shape[0]
    W_big = jnp.zeros((D, 2 * EMB), jnp.float32)
    W_big = W_big.at[2:2 + N_GENRE, 0:EMB].set(W_genre)
    W_big = W_big.at[2 + N_GENRE:, EMB:].set(W_director)

    return pl.pallas_call(
        _tile_kernel,
        grid=(B // BT,),
        in_specs=[
            pl.BlockSpec((BT, D), lambda i: (i, 0)),
            pl.BlockSpec((D, 2 * EMB), lambda i: (0, 0)),
            pl.BlockSpec((N_RATE, EMB), lambda i: (0, 0)),
            pl.BlockSpec((N_YEAR, EMB), lambda i: (0, 0)),
        ],
        out_specs=pl.BlockSpec((BT, 4 * EMB), lambda i: (i, 0)),
        out_shape=jax.ShapeDtypeStruct((B, 4 * EMB), jnp.float32),
    )(x, W_big, W_rate, W_year)


# speedup vs baseline: 1.0070x; 1.0070x over previous
import jax
import jax.numpy as jnp
from jax.experimental import pallas as pl

N_RATE = 6
N_GENRE = 25
N_DIRECTOR = 2186
N_YEAR = 81
EMB = 32
D = 2 + N_GENRE + N_DIRECTOR
BT = 1024


def _tile_kernel(x_ref, w_big_ref, w_rate_ref, w_year_ref, out_ref):
    xf = x_ref[...].astype(jnp.float32)
    big = jax.lax.dot_general(
        xf, w_big_ref[...],
        (((1,), (0,)), ((), ())),
        preferred_element_type=jnp.float32,
    )
    gd = jax.nn.sigmoid(big)

    rate_idx = x_ref[:, 0:1]
    year_idx = x_ref[:, 1:2]
    oh_rate = (rate_idx == jax.lax.broadcasted_iota(jnp.int32, (1, N_RATE), 1)
               ).astype(jnp.float32)
    oh_year = (year_idx == jax.lax.broadcasted_iota(jnp.int32, (1, N_YEAR), 1)
               ).astype(jnp.float32)
    rate_emb = jax.lax.dot_general(
        oh_rate, w_rate_ref[...], (((1,), (0,)), ((), ())),
        preferred_element_type=jnp.float32)
    year_emb = jax.lax.dot_general(
        oh_year, w_year_ref[...], (((1,), (0,)), ((), ())),
        preferred_element_type=jnp.float32)

    out_ref[...] = jnp.concatenate([rate_emb, year_emb, gd], axis=1)


def kernel(x, W_rate, W_year, W_genre, W_director):
    B = x.shape[0]
    W_big = jnp.zeros((D, 2 * EMB), jnp.float32)
    W_big = W_big.at[2:2 + N_GENRE, 0:EMB].set(W_genre)
    W_big = W_big.at[2 + N_GENRE:, EMB:].set(W_director)

    return pl.pallas_call(
        _tile_kernel,
        grid=(B // BT,),
        in_specs=[
            pl.BlockSpec((BT, D), lambda i: (i, 0)),
            pl.BlockSpec((D, 2 * EMB), lambda i: (0, 0)),
            pl.BlockSpec((N_RATE, EMB), lambda i: (0, 0)),
            pl.BlockSpec((N_YEAR, EMB), lambda i: (0, 0)),
        ],
        out_specs=pl.BlockSpec((BT, 4 * EMB), lambda i: (i, 0)),
        out_shape=jax.ShapeDtypeStruct((B, 4 * EMB), jnp.float32),
    )(x, W_big, W_rate, W_year)
